# baseline (speedup 1.0000x reference)
import functools

import jax
import jax.numpy as jnp
from jax import lax
from jax.experimental import pallas as pl
from jax.experimental.pallas import tpu as pltpu
from jax.experimental.pallas import tpu_sc as plsc

BIGRAM_VOCAB = 100000
HID = 128
MODEL_DIM = 1024
BATCH = 4
SEQLEN = 4096
TOK = BATCH * SEQLEN

NC, NS = 2, 16
NW = NC * NS
GSTREAM = 64


def _make_sc_gather(ctok):
    chunk = ctok // NW
    ng = -(-chunk // GSTREAM)
    gs = chunk // ng
    w_per_row = SEQLEN // chunk

    @functools.partial(
        pl.kernel,
        mesh=plsc.VectorSubcoreMesh(core_axis_name="c", subcore_axis_name="s"),
        out_type=jax.ShapeDtypeStruct((ctok, HID), jnp.float32),
        compiler_params=pltpu.CompilerParams(skip_device_barrier=True),
        scratch_types=[
            pltpu.VMEM((chunk + 16,), jnp.int32),
            pltpu.VMEM((chunk,), jnp.int32),
            pltpu.VMEM((chunk, HID), jnp.float32),
            pltpu.SemaphoreType.DMA,
            pltpu.SemaphoreType.DMA,
            pltpu.SemaphoreType.DMA,
        ],
    )
    def sc_gather(pad_hbm, table_hbm, h_hbm, ids_v, idx_v, rows_v,
                  sem_in, sem_g, sem_w):
        wid = lax.axis_index("s") * NC + lax.axis_index("c")
        base = wid * chunk
        pltpu.async_copy(pad_hbm.at[pl.ds(base, chunk + 16)], ids_v,
                         sem_in).wait()
        lane = lax.iota(jnp.int32, 16)
        keep0 = jnp.broadcast_to(
            jnp.minimum(lax.rem(wid, jnp.int32(w_per_row)), jnp.int32(1)),
            (16,))
        vecs_per_g = gs // 16

        def fire_body(j, carry):
            for v in range(vecs_per_g):
                i = j * vecs_per_g + v
                cur = ids_v[pl.ds(16 + i * 16, 16)]
                prev = ids_v[pl.ds(15 + i * 16, 16)]
                keep = jnp.minimum(
                    lane + keep0 + jnp.minimum(i, jnp.int32(1)), jnp.int32(1))
                h = lax.rem(prev * keep * 3 + cur, jnp.int32(100000))
                idx_v[pl.ds(i * 16, 16)] = h
            off = pl.multiple_of(j * gs, 8)
            pltpu.async_copy(table_hbm.at[idx_v.at[pl.ds(off, gs)]],
                             rows_v.at[pl.ds(off, gs)], sem_g)
            return carry
        lax.fori_loop(jnp.int32(0), jnp.int32(ng), fire_body, jnp.int32(0))

        def drain_body(j, carry):
            pltpu.make_async_copy(table_hbm.at[idx_v.at[pl.ds(0, gs)]],
                                  rows_v.at[pl.ds(0, gs)], sem_g).wait()
            off = pl.multiple_of(j * gs, 8)
            pltpu.async_copy(rows_v.at[pl.ds(off, gs)],
                             h_hbm.at[pl.ds(base + off, gs)], sem_w)
            return carry
        lax.fori_loop(jnp.int32(0), jnp.int32(ng), drain_body, jnp.int32(0))

        def drainw_body(j, carry):
            pltpu.make_async_copy(rows_v.at[pl.ds(0, gs)],
                                  h_hbm.at[pl.ds(0, gs)], sem_w).wait()
            return carry
        lax.fori_loop(jnp.int32(0), jnp.int32(ng), drainw_body, jnp.int32(0))

    return sc_gather


_sc_gather_chunk = _make_sc_gather(TOK)


def _proj_body(h_ref, w_ref, o_ref):
    o_ref[...] = lax.dot_general(
        h_ref[...], w_ref[...], (((1,), (1,)), ((), ())),
        preferred_element_type=jnp.float32)


_ROWS_BLK = 2048


def _tc_project(h, Wproj):
    rows = h.shape[0]
    return pl.pallas_call(
        _proj_body,
        grid=(rows // _ROWS_BLK,),
        in_specs=[
            pl.BlockSpec((_ROWS_BLK, HID), lambda i: (i, jnp.int32(0))),
            pl.BlockSpec((MODEL_DIM, HID),
                         lambda i: (jnp.int32(0), jnp.int32(0))),
        ],
        out_specs=pl.BlockSpec((_ROWS_BLK, MODEL_DIM),
                               lambda i: (i, jnp.int32(0))),
        out_shape=jax.ShapeDtypeStruct((rows, MODEL_DIM), jnp.float32),
    )(h, Wproj)


def kernel(input_ids, table, Wproj):
    padded = jnp.concatenate(
        [jnp.zeros((16,), jnp.int32), input_ids.reshape(TOK).astype(jnp.int32)])
    h = _sc_gather_chunk(padded, table)
    out = _tc_project(h, Wproj)
    return out.reshape(BATCH, SEQLEN, MODEL_DIM)


# speedup vs baseline: 1.0147x; 1.0147x over previous
import functools

import jax
import jax.numpy as jnp
from jax import lax
from jax.experimental import pallas as pl
from jax.experimental.pallas import tpu as pltpu
from jax.experimental.pallas import tpu_sc as plsc

BIGRAM_VOCAB = 100000
HID = 128
MODEL_DIM = 1024
BATCH = 4
SEQLEN = 4096
TOK = BATCH * SEQLEN

NC, NS = 2, 16
NW = NC * NS
GSTREAM = 16


def _make_sc_gather(ctok):
    chunk = ctok // NW
    ng = -(-chunk // GSTREAM)
    gs = chunk // ng
    w_per_row = SEQLEN // chunk

    @functools.partial(
        pl.kernel,
        mesh=plsc.VectorSubcoreMesh(core_axis_name="c", subcore_axis_name="s"),
        out_type=jax.ShapeDtypeStruct((ctok, HID), jnp.float32),
        compiler_params=pltpu.CompilerParams(skip_device_barrier=True),
        scratch_types=[
            pltpu.VMEM((chunk + 16,), jnp.int32),
            pltpu.VMEM((chunk,), jnp.int32),
            pltpu.VMEM((chunk, HID), jnp.float32),
            pltpu.SemaphoreType.DMA,
            pltpu.SemaphoreType.DMA,
            pltpu.SemaphoreType.DMA,
        ],
    )
    def sc_gather(pad_hbm, table_hbm, h_hbm, ids_v, idx_v, rows_v,
                  sem_in, sem_g, sem_w):
        wid = lax.axis_index("s") * NC + lax.axis_index("c")
        base = wid * chunk
        pltpu.async_copy(pad_hbm.at[pl.ds(base, chunk + 16)], ids_v,
                         sem_in).wait()
        lane = lax.iota(jnp.int32, 16)
        keep0 = jnp.broadcast_to(
            jnp.minimum(lax.rem(wid, jnp.int32(w_per_row)), jnp.int32(1)),
            (16,))
        vecs_per_g = gs // 16

        def fire_body(j, carry):
            for v in range(vecs_per_g):
                i = j * vecs_per_g + v
                cur = ids_v[pl.ds(16 + i * 16, 16)]
                prev = ids_v[pl.ds(15 + i * 16, 16)]
                keep = jnp.minimum(
                    lane + keep0 + jnp.minimum(i, jnp.int32(1)), jnp.int32(1))
                h = lax.rem(prev * keep * 3 + cur, jnp.int32(100000))
                idx_v[pl.ds(i * 16, 16)] = h
            off = pl.multiple_of(j * gs, 8)
            pltpu.async_copy(table_hbm.at[idx_v.at[pl.ds(off, gs)]],
                             rows_v.at[pl.ds(off, gs)], sem_g)
            return carry
        lax.fori_loop(jnp.int32(0), jnp.int32(ng), fire_body, jnp.int32(0))

        def drain_body(j, carry):
            pltpu.make_async_copy(table_hbm.at[idx_v.at[pl.ds(0, gs)]],
                                  rows_v.at[pl.ds(0, gs)], sem_g).wait()
            off = pl.multiple_of(j * gs, 8)
            pltpu.async_copy(rows_v.at[pl.ds(off, gs)],
                             h_hbm.at[pl.ds(base + off, gs)], sem_w)
            return carry
        lax.fori_loop(jnp.int32(0), jnp.int32(ng), drain_body, jnp.int32(0))

        def drainw_body(j, carry):
            pltpu.make_async_copy(rows_v.at[pl.ds(0, gs)],
                                  h_hbm.at[pl.ds(0, gs)], sem_w).wait()
            return carry
        lax.fori_loop(jnp.int32(0), jnp.int32(ng), drainw_body, jnp.int32(0))

    return sc_gather


_sc_gather_chunk = _make_sc_gather(TOK)


def _proj_body(h_ref, w_ref, o_ref):
    o_ref[...] = lax.dot_general(
        h_ref[...], w_ref[...], (((1,), (1,)), ((), ())),
        preferred_element_type=jnp.float32)


_ROWS_BLK = 2048


def _tc_project(h, Wproj):
    rows = h.shape[0]
    return pl.pallas_call(
        _proj_body,
        grid=(rows // _ROWS_BLK,),
        in_specs=[
            pl.BlockSpec((_ROWS_BLK, HID), lambda i: (i, jnp.int32(0))),
            pl.BlockSpec((MODEL_DIM, HID),
                         lambda i: (jnp.int32(0), jnp.int32(0))),
        ],
        out_specs=pl.BlockSpec((_ROWS_BLK, MODEL_DIM),
                               lambda i: (i, jnp.int32(0))),
        out_shape=jax.ShapeDtypeStruct((rows, MODEL_DIM), jnp.float32),
    )(h, Wproj)


def kernel(input_ids, table, Wproj):
    padded = jnp.concatenate(
        [jnp.zeros((16,), jnp.int32), input_ids.reshape(TOK).astype(jnp.int32)])
    h = _sc_gather_chunk(padded, table)
    out = _tc_project(h, Wproj)
    return out.reshape(BATCH, SEQLEN, MODEL_DIM)
